# baseline (speedup 1.0000x reference)
import functools

import jax
import jax.numpy as jnp
from jax import lax
from jax.experimental import pallas as pl
from jax.experimental.pallas import tpu as pltpu
from jax.experimental.pallas import tpu_sc as plsc

B, S, H = 4, 8192, 1024
NC, NS, L = 2, 16, 16
NW = NC * NS
S_PER_W = S // NW
CS = 16
CHUNKS = S_PER_W // CS
CW = CS * H


@functools.cache
def _build():
    mesh = plsc.VectorSubcoreMesh(
        core_axis_name="c", subcore_axis_name="s", num_cores=NC, num_subcores=NS
    )

    @functools.partial(
        pl.kernel,
        out_type=jax.ShapeDtypeStruct((B, S * H), jnp.float32),
        mesh=mesh,
        scratch_types=[
            pltpu.VMEM((CW,), jnp.float32),
            [pltpu.VMEM((CW,), jnp.float32) for _ in range(B)],
        ],
    )
    def _pos_add(x_hbm, t_hbm, o_hbm, t_v, x_vs):
        wid = lax.axis_index("s") * NC + lax.axis_index("c")
        base = wid * (S_PER_W * H)

        @pl.loop(0, CHUNKS)
        def _chunk(c):
            off = base + c * CW
            pltpu.sync_copy(t_hbm.at[pl.ds(off, CW)], t_v)
            for b in range(B):
                pltpu.sync_copy(x_hbm.at[b, pl.ds(off, CW)], x_vs[b])

            @pl.loop(0, CW, step=L, unroll=4)
            def _grp(i):
                t = t_v[pl.ds(i, L)]
                for b in range(B):
                    x_vs[b][pl.ds(i, L)] = x_vs[b][pl.ds(i, L)] + t

            for b in range(B):
                pltpu.sync_copy(x_vs[b], o_hbm.at[b, pl.ds(off, CW)])

    return _pos_add


def kernel(x, pos_table):
    xf = x.reshape(B, S * H)
    tf = pos_table[:S].reshape(S * H)
    out = _build()(xf, tf)
    return out.reshape(B, S, H)
